# baseline (speedup 1.0000x reference)
import functools

import jax
import jax.numpy as jnp
from jax import lax
from jax.experimental import pallas as pl
from jax.experimental.pallas import tpu as pltpu
from jax.experimental.pallas import tpu_sc as plsc

EMBED = 32
NC, NS = 2, 16
NW = NC * NS
BW = 128
BWP = BW + 1
K = 2
L = 16


def _format_sc(tableT, tail16):
    E, V = tableT.shape
    n_full = V // BW
    tail = V - n_full * BW
    slots = 246
    mesh = plsc.VectorSubcoreMesh(core_axis_name="c", subcore_axis_name="s")

    @functools.partial(
        pl.kernel,
        out_type=jax.ShapeDtypeStruct((V * E // BW, BW), jnp.float32),
        mesh=mesh,
        compiler_params=pltpu.CompilerParams(
            use_tc_tiling_on_sc=True, needs_layout_passes=False),
        scratch_types=(
            [pltpu.VMEM((E, BW), jnp.float32) for _ in range(K)]
            + [pltpu.VMEM((BW * (E + 1),), jnp.float32) for _ in range(K)]
            + [pltpu.VMEM((E, BW), jnp.float32) for _ in range(K)]
            + [pltpu.SemaphoreType.DMA for _ in range(2 * K)]
        ),
    )
    def k(tableT_hbm, tail_hbm, out_hbm, *scr):
        sbufs = scr[:K]
        dbvs = scr[K:2 * K]
        dbrs = scr[2 * K:3 * K]
        gsem = scr[3 * K:4 * K]
        wsem = scr[4 * K:]
        wid = lax.axis_index("s") * NC + lax.axis_index("c")
        lanes = lax.iota(jnp.int32, L)
        i33 = lanes * (E + 1)

        def vt_of(j):
            return j * NW + wid

        def in_bounds(j):
            return vt_of(j) < n_full

        def fire_in(b, j):
            def _():
                pltpu.async_copy(
                    tableT_hbm.at[:, pl.ds(vt_of(j) * BW, BW)],
                    sbufs[b], gsem[b])
            pl.when(in_bounds(j))(_)

        def wait_in(b, j):
            def _():
                pltpu.make_async_copy(
                    tableT_hbm.at[:, pl.ds(0, BW)], sbufs[b], gsem[b]).wait()
            pl.when(in_bounds(j))(_)

        def drain_out(b, j):
            def _():
                pltpu.make_async_copy(
                    dbrs[b], out_hbm.at[pl.ds(0, E)], wsem[b]).wait()
            pl.when(in_bounds(j))(_)

        def transpose(b):
            for e in range(E):
                for v0 in range(0, BW, L):
                    vec = sbufs[b][e, pl.ds(v0, L)]
                    plsc.store_scatter(
                        dbvs[b], [i33 + (v0 * (E + 1) + e)], vec)
            for r in range(E):
                for sv in range(4):
                    v = 4 * r + sv
                    for h in range(2):
                        vec = plsc.load_gather(
                            dbvs[b], [lanes + (v * (E + 1) + h * L)])
                        dbrs[b][r, pl.ds(sv * E + h * L, L)] = vec

        @pl.when(wid == 0)
        def _():
            pltpu.sync_copy(tail_hbm, sbufs[0].at[pl.ds(0, 16)])
            pltpu.sync_copy(
                sbufs[0].at[pl.ds(0, 16)],
                out_hbm.at[pl.ds(n_full * BW * E // BW, 16)])

        for b in range(K):
            fire_in(b, b)

        def body(i, carry):
            for b in range(K):
                j = 2 * i + b
                wait_in(b, j)
                pl.when(jnp.logical_and(j >= 2, in_bounds(j - 2)))(
                    lambda b=b: pltpu.make_async_copy(
                        dbrs[b], out_hbm.at[pl.ds(0, E)], wsem[b]).wait())

                def work(b=b, j=j):
                    transpose(b)
                    pltpu.async_copy(
                        dbrs[b], out_hbm.at[pl.ds(vt_of(j) * E, E)], wsem[b])
                pl.when(in_bounds(j))(work)
                fire_in(b, j + 2)
            return carry

        lax.fori_loop(0, slots // 2, body, 0)
        for b in range(K):
            drain_out(b, slots - 2 + b)

    return k(tableT, tail16)


@jax.jit
def _gather_sc(xT, table):
    length, batch = xT.shape
    assert batch == NW * BW
    n_iters = length // K
    assert length % K == 0
    mesh = plsc.VectorSubcoreMesh(core_axis_name="c", subcore_axis_name="s")

    @functools.partial(
        pl.kernel,
        out_type=jax.ShapeDtypeStruct(
            (length, EMBED // 8, NW, 8, BW), jnp.float32),
        mesh=mesh,
        compiler_params=pltpu.CompilerParams(
            use_tc_tiling_on_sc=False, needs_layout_passes=False),
        scratch_types=(
            [pltpu.VMEM((length, BW), jnp.int32)]
            + [pltpu.VMEM((BW, EMBED), jnp.float32) for _ in range(K)]
            + [pltpu.VMEM((EMBED // 8, 8, BWP), jnp.float32) for _ in range(K)]
            + [pltpu.SemaphoreType.DMA for _ in range(2 * K)]
        ),
    )
    def k(xT_hbm, table_hbm, out_hbm, idx_v, *bufs_sems):
        bufs = bufs_sems[:K]
        tbufs = bufs_sems[K:2 * K]
        gsem = bufs_sems[2 * K:3 * K]
        wsem = bufs_sems[3 * K:]
        wid = lax.axis_index("s") * NC + lax.axis_index("c")
        b0 = wid * BW
        pltpu.sync_copy(xT_hbm.at[:, pl.ds(b0, BW)], idx_v)

        lanes = lax.iota(jnp.int32, L)
        ebs = [(lanes + e0) // 8 for e0 in (0, L)]
        ess = [(lanes + e0) % 8 for e0 in (0, L)]

        def tbuf_window(b):
            return tbufs[b].at[:, :, pl.ds(0, BW)]

        def fire_gather(b, t):
            return pltpu.async_copy(
                table_hbm.at[idx_v.at[t]], bufs[b], gsem[b])

        for b in range(K):
            fire_gather(b, b)

        def body(i, carry):
            for b in range(K):
                t = i * K + b
                pltpu.make_async_copy(
                    table_hbm.at[idx_v.at[0]], bufs[b], gsem[b]).wait()

                def drain_tbuf(b=b):
                    pltpu.make_async_copy(
                        tbuf_window(b), out_hbm.at[0, :, wid], wsem[b]).wait()
                pl.when(i != 0)(drain_tbuf)
                for v in range(BW):
                    vcol = jnp.full((L,), v, jnp.int32)
                    for h in range(2):
                        vec = bufs[b][v, pl.ds(h * L, L)]
                        plsc.store_scatter(
                            tbufs[b], [ebs[h], ess[h], vcol], vec)
                pltpu.async_copy(
                    tbuf_window(b), out_hbm.at[t, :, wid], wsem[b])

                @pl.when(t + K < length)
                def _():
                    fire_gather(b, t + K)
            return carry

        lax.fori_loop(0, n_iters, body, 0)
        for b in range(K):
            pltpu.make_async_copy(
                tbuf_window(b), out_hbm.at[0, :, wid], wsem[b]).wait()

    return k(xT, table)


@jax.jit
def _impl(x, idx2vec):
    batch, length = x.shape
    vocab = idx2vec.shape[0]
    tail16 = idx2vec[(vocab // BW) * BW:].reshape(-1, BW)
    scratch = _format_sc(idx2vec.T, tail16)
    out5 = _gather_sc(x.T, scratch.reshape(vocab, EMBED))
    return out5.transpose(2, 4, 0, 1, 3).reshape(batch, length, EMBED)


def kernel(x, idx2vec):
    return _impl(x, idx2vec)


# speedup vs baseline: 2.7124x; 2.7124x over previous
import functools

import jax
import jax.numpy as jnp
from jax import lax
from jax.experimental import pallas as pl
from jax.experimental.pallas import tpu as pltpu
from jax.experimental.pallas import tpu_sc as plsc

EMBED = 32
NC, NS = 2, 16
NW = NC * NS
BW = 128
BWP = BW + 1
K = 2
L = 16


def _format_sc(tableT, tail16):
    E, V = tableT.shape
    n_full = V // BW
    tail = V - n_full * BW
    slots = 246
    mesh = plsc.VectorSubcoreMesh(core_axis_name="c", subcore_axis_name="s")

    @functools.partial(
        pl.kernel,
        out_type=jax.ShapeDtypeStruct((V * E // BW, BW), jnp.float32),
        mesh=mesh,
        compiler_params=pltpu.CompilerParams(
            use_tc_tiling_on_sc=True, needs_layout_passes=False),
        scratch_types=(
            [pltpu.VMEM((E, BW), jnp.float32) for _ in range(K)]
            + [pltpu.VMEM((BW * (E + 1),), jnp.float32) for _ in range(K)]
            + [pltpu.VMEM((E, BW), jnp.float32) for _ in range(K)]
            + [pltpu.SemaphoreType.DMA for _ in range(2 * K)]
        ),
    )
    def k(tableT_hbm, tail_hbm, out_hbm, *scr):
        sbufs = scr[:K]
        dbvs = scr[K:2 * K]
        dbrs = scr[2 * K:3 * K]
        gsem = scr[3 * K:4 * K]
        wsem = scr[4 * K:]
        wid = lax.axis_index("s") * NC + lax.axis_index("c")
        lanes = lax.iota(jnp.int32, L)
        i33 = lanes * (E + 1)

        def vt_of(j):
            return j * NW + wid

        def in_bounds(j):
            return vt_of(j) < n_full

        def fire_in(b, j):
            def _():
                pltpu.async_copy(
                    tableT_hbm.at[:, pl.ds(vt_of(j) * BW, BW)],
                    sbufs[b], gsem[b])
            pl.when(in_bounds(j))(_)

        def wait_in(b, j):
            def _():
                pltpu.make_async_copy(
                    tableT_hbm.at[:, pl.ds(0, BW)], sbufs[b], gsem[b]).wait()
            pl.when(in_bounds(j))(_)

        def drain_out(b, j):
            def _():
                pltpu.make_async_copy(
                    dbrs[b], out_hbm.at[pl.ds(0, E)], wsem[b]).wait()
            pl.when(in_bounds(j))(_)

        def transpose(b):
            @plsc.parallel_loop(0, E, unroll=8)
            def _(e):
                for v0 in range(0, BW, L):
                    vec = sbufs[b][e, pl.ds(v0, L)]
                    plsc.store_scatter(
                        dbvs[b], [i33 + (v0 * (E + 1) + e)], vec)
            @plsc.parallel_loop(0, E, unroll=8)
            def _(r):
                for sv in range(4):
                    for h in range(2):
                        vec = plsc.load_gather(
                            dbvs[b],
                            [lanes + ((4 * r + sv) * (E + 1) + h * L)])
                        dbrs[b][r, pl.ds(sv * E + h * L, L)] = vec

        @pl.when(wid == 0)
        def _():
            pltpu.sync_copy(tail_hbm, sbufs[0].at[pl.ds(0, 16)])
            pltpu.sync_copy(
                sbufs[0].at[pl.ds(0, 16)],
                out_hbm.at[pl.ds(n_full * BW * E // BW, 16)])

        for b in range(K):
            fire_in(b, b)

        def body(i, carry):
            for b in range(K):
                j = 2 * i + b
                wait_in(b, j)
                pl.when(jnp.logical_and(j >= 2, in_bounds(j - 2)))(
                    lambda b=b: pltpu.make_async_copy(
                        dbrs[b], out_hbm.at[pl.ds(0, E)], wsem[b]).wait())

                def work(b=b, j=j):
                    transpose(b)
                    pltpu.async_copy(
                        dbrs[b], out_hbm.at[pl.ds(vt_of(j) * E, E)], wsem[b])
                pl.when(in_bounds(j))(work)
                fire_in(b, j + 2)
            return carry

        lax.fori_loop(0, slots // 2, body, 0)
        for b in range(K):
            drain_out(b, slots - 2 + b)

    return k(tableT, tail16)


@jax.jit
def _gather_sc(xT, table):
    length, batch = xT.shape
    assert batch == NW * BW
    n_iters = length // K
    assert length % K == 0
    mesh = plsc.VectorSubcoreMesh(core_axis_name="c", subcore_axis_name="s")

    @functools.partial(
        pl.kernel,
        out_type=jax.ShapeDtypeStruct(
            (length, EMBED // 8, NW, 8, BW), jnp.float32),
        mesh=mesh,
        compiler_params=pltpu.CompilerParams(
            use_tc_tiling_on_sc=False, needs_layout_passes=False),
        scratch_types=(
            [pltpu.VMEM((length, BW), jnp.int32)]
            + [pltpu.VMEM((BW, EMBED), jnp.float32) for _ in range(K)]
            + [pltpu.VMEM((EMBED // 8, 8, BWP), jnp.float32) for _ in range(K)]
            + [pltpu.SemaphoreType.DMA for _ in range(2 * K)]
        ),
    )
    def k(xT_hbm, table_hbm, out_hbm, idx_v, *bufs_sems):
        bufs = bufs_sems[:K]
        tbufs = bufs_sems[K:2 * K]
        gsem = bufs_sems[2 * K:3 * K]
        wsem = bufs_sems[3 * K:]
        wid = lax.axis_index("s") * NC + lax.axis_index("c")
        b0 = wid * BW
        pltpu.sync_copy(xT_hbm.at[:, pl.ds(b0, BW)], idx_v)

        lanes = lax.iota(jnp.int32, L)
        ebs = [(lanes + e0) // 8 for e0 in (0, L)]
        ess = [(lanes + e0) % 8 for e0 in (0, L)]

        def tbuf_window(b):
            return tbufs[b].at[:, :, pl.ds(0, BW)]

        def fire_gather(b, t):
            return pltpu.async_copy(
                table_hbm.at[idx_v.at[t]], bufs[b], gsem[b])

        for b in range(K):
            fire_gather(b, b)

        def body(i, carry):
            for b in range(K):
                t = i * K + b
                pltpu.make_async_copy(
                    table_hbm.at[idx_v.at[0]], bufs[b], gsem[b]).wait()

                def drain_tbuf(b=b):
                    pltpu.make_async_copy(
                        tbuf_window(b), out_hbm.at[0, :, wid], wsem[b]).wait()
                pl.when(i != 0)(drain_tbuf)
                @plsc.parallel_loop(0, BW, unroll=8)
                def _(v):
                    vcol = jnp.full((L,), v, jnp.int32)
                    for h in range(2):
                        vec = bufs[b][v, pl.ds(h * L, L)]
                        plsc.store_scatter(
                            tbufs[b], [ebs[h], ess[h], vcol], vec)
                pltpu.async_copy(
                    tbuf_window(b), out_hbm.at[t, :, wid], wsem[b])

                @pl.when(t + K < length)
                def _():
                    fire_gather(b, t + K)
            return carry

        lax.fori_loop(0, n_iters, body, 0)
        for b in range(K):
            pltpu.make_async_copy(
                tbuf_window(b), out_hbm.at[0, :, wid], wsem[b]).wait()

    return k(xT, table)


@jax.jit
def _impl(x, idx2vec):
    batch, length = x.shape
    vocab = idx2vec.shape[0]
    tail16 = idx2vec[(vocab // BW) * BW:].reshape(-1, BW)
    scratch = _format_sc(idx2vec.T, tail16)
    out5 = _gather_sc(x.T, scratch.reshape(vocab, EMBED))
    return out5.transpose(2, 4, 0, 1, 3).reshape(batch, length, EMBED)


def kernel(x, idx2vec):
    return _impl(x, idx2vec)


# speedup vs baseline: 3.8238x; 1.4097x over previous
import functools

import jax
import jax.numpy as jnp
from jax import lax
from jax.experimental import pallas as pl
from jax.experimental.pallas import tpu as pltpu
from jax.experimental.pallas import tpu_sc as plsc

EMBED = 32
NC, NS = 2, 16
NW = NC * NS
BW = 128
BWP = BW + 1
K = 2
KG = 4
L = 16


def _format_sc(tableT, tail16):
    E, V = tableT.shape
    n_full = V // BW
    SW = 2
    n_sup = n_full // SW
    slots = 124
    mesh = plsc.VectorSubcoreMesh(core_axis_name="c", subcore_axis_name="s")

    @functools.partial(
        pl.kernel,
        out_type=jax.ShapeDtypeStruct((V * E // BW, BW), jnp.float32),
        mesh=mesh,
        compiler_params=pltpu.CompilerParams(
            use_tc_tiling_on_sc=True, needs_layout_passes=False),
        scratch_types=(
            [pltpu.VMEM((E, SW * BW), jnp.float32) for _ in range(K)]
            + [pltpu.VMEM((SW * BW * (E + 1),), jnp.float32) for _ in range(K)]
            + [pltpu.VMEM((SW * E, BW), jnp.float32) for _ in range(K)]
            + [pltpu.SemaphoreType.DMA for _ in range(2 * K)]
        ),
    )
    def k(tableT_hbm, tail_hbm, out_hbm, *scr):
        sbufs = scr[:K]
        dbvs = scr[K:2 * K]
        dbrs = scr[2 * K:3 * K]
        gsem = scr[3 * K:4 * K]
        wsem = scr[4 * K:]
        wid = lax.axis_index("s") * NC + lax.axis_index("c")
        lanes = lax.iota(jnp.int32, L)
        i33 = lanes * (E + 1)

        def s_of(j):
            return j * NW + wid

        def in_bounds(j):
            return s_of(j) < n_sup

        def fire_in(b, j):
            def _():
                pltpu.async_copy(
                    tableT_hbm.at[:, pl.ds(s_of(j) * SW * BW, SW * BW)],
                    sbufs[b], gsem[b])
            pl.when(in_bounds(j))(_)

        def wait_in(b, j):
            def _():
                pltpu.make_async_copy(
                    tableT_hbm.at[:, pl.ds(0, SW * BW)],
                    sbufs[b], gsem[b]).wait()
            pl.when(in_bounds(j))(_)

        def drain_out(b, j):
            def _():
                pltpu.make_async_copy(
                    dbrs[b], out_hbm.at[pl.ds(0, SW * E)], wsem[b]).wait()
            pl.when(in_bounds(j))(_)

        def transpose(b):
            @plsc.parallel_loop(0, E, unroll=8)
            def _(e):
                for kk in range(SW):
                    for v0 in range(0, BW, L):
                        vec = sbufs[b][e, pl.ds(kk * BW + v0, L)]
                        plsc.store_scatter(
                            dbvs[b],
                            [i33 + (kk * BW * (E + 1) + v0 * (E + 1) + e)],
                            vec)
            @plsc.parallel_loop(0, SW * E, unroll=8)
            def _(r):
                kk = r // E
                rr = r - kk * E
                base = kk * BW * (E + 1)
                for sv in range(4):
                    for h in range(2):
                        vec = plsc.load_gather(
                            dbvs[b],
                            [lanes + (base + (4 * rr + sv) * (E + 1) + h * L)])
                        dbrs[b][r, pl.ds(sv * E + h * L, L)] = vec

        @pl.when(wid == 0)
        def _():
            pltpu.sync_copy(tail_hbm, sbufs[0].at[pl.ds(0, 16), pl.ds(0, BW)])
            pltpu.sync_copy(
                sbufs[0].at[pl.ds(0, 16), pl.ds(0, BW)],
                out_hbm.at[pl.ds(n_full * BW * E // BW, 16)])

        for b in range(K):
            fire_in(b, b)

        def body(i, carry):
            for b in range(K):
                j = 2 * i + b
                wait_in(b, j)
                pl.when(jnp.logical_and(j >= 2, in_bounds(j - 2)))(
                    lambda b=b: pltpu.make_async_copy(
                        dbrs[b], out_hbm.at[pl.ds(0, SW * E)], wsem[b]).wait())

                def work(b=b, j=j):
                    transpose(b)
                    pltpu.async_copy(
                        dbrs[b],
                        out_hbm.at[pl.ds(s_of(j) * SW * E, SW * E)], wsem[b])
                pl.when(in_bounds(j))(work)
                fire_in(b, j + 2)
            return carry

        lax.fori_loop(0, slots // 2, body, 0)
        for b in range(K):
            drain_out(b, slots - 2 + b)

    return k(tableT, tail16)


@jax.jit
def _gather_sc(xT, table):
    length, batch = xT.shape
    assert batch == NW * BW
    n_iters = length // KG
    assert length % KG == 0
    mesh = plsc.VectorSubcoreMesh(core_axis_name="c", subcore_axis_name="s")

    @functools.partial(
        pl.kernel,
        out_type=jax.ShapeDtypeStruct(
            (length, EMBED // 8, NW, 8, BW), jnp.float32),
        mesh=mesh,
        compiler_params=pltpu.CompilerParams(
            use_tc_tiling_on_sc=False, needs_layout_passes=False),
        scratch_types=(
            [pltpu.VMEM((length, BW), jnp.int32)]
            + [pltpu.VMEM((BW, EMBED), jnp.float32) for _ in range(KG)]
            + [pltpu.VMEM((EMBED // 8, 8, BWP), jnp.float32) for _ in range(KG)]
            + [pltpu.SemaphoreType.DMA for _ in range(2 * KG)]
        ),
    )
    def k(xT_hbm, table_hbm, out_hbm, idx_v, *bufs_sems):
        bufs = bufs_sems[:KG]
        tbufs = bufs_sems[KG:2 * KG]
        gsem = bufs_sems[2 * KG:3 * KG]
        wsem = bufs_sems[3 * KG:]
        wid = lax.axis_index("s") * NC + lax.axis_index("c")
        b0 = wid * BW
        pltpu.sync_copy(xT_hbm.at[:, pl.ds(b0, BW)], idx_v)

        lanes = lax.iota(jnp.int32, L)
        ebs = [(lanes + e0) // 8 for e0 in (0, L)]
        ess = [(lanes + e0) % 8 for e0 in (0, L)]

        def tbuf_window(b):
            return tbufs[b].at[:, :, pl.ds(0, BW)]

        def fire_gather(b, t):
            return pltpu.async_copy(
                table_hbm.at[idx_v.at[t]], bufs[b], gsem[b])

        for b in range(KG):
            fire_gather(b, b)

        def body(i, carry):
            for b in range(KG):
                t = i * KG + b
                pltpu.make_async_copy(
                    table_hbm.at[idx_v.at[0]], bufs[b], gsem[b]).wait()

                def drain_tbuf(b=b):
                    pltpu.make_async_copy(
                        tbuf_window(b), out_hbm.at[0, :, wid], wsem[b]).wait()
                pl.when(i != 0)(drain_tbuf)
                @plsc.parallel_loop(0, BW, unroll=8)
                def _(v):
                    vcol = jnp.full((L,), v, jnp.int32)
                    for h in range(2):
                        vec = bufs[b][v, pl.ds(h * L, L)]
                        plsc.store_scatter(
                            tbufs[b], [ebs[h], ess[h], vcol], vec)
                pltpu.async_copy(
                    tbuf_window(b), out_hbm.at[t, :, wid], wsem[b])

                @pl.when(t + KG < length)
                def _():
                    fire_gather(b, t + KG)
            return carry

        lax.fori_loop(0, n_iters, body, 0)
        for b in range(KG):
            pltpu.make_async_copy(
                tbuf_window(b), out_hbm.at[0, :, wid], wsem[b]).wait()

    return k(xT, table)


@jax.jit
def _impl(x, idx2vec):
    batch, length = x.shape
    vocab = idx2vec.shape[0]
    tail16 = idx2vec[(vocab // BW) * BW:].reshape(-1, BW)
    scratch = _format_sc(idx2vec.T, tail16)
    out5 = _gather_sc(x.T, scratch.reshape(vocab, EMBED))
    return out5.transpose(2, 4, 0, 1, 3).reshape(batch, length, EMBED)


def kernel(x, idx2vec):
    return _impl(x, idx2vec)


# speedup vs baseline: 4.0963x; 1.0713x over previous
import functools

import jax
import jax.numpy as jnp
from jax import lax
from jax.experimental import pallas as pl
from jax.experimental.pallas import tpu as pltpu
from jax.experimental.pallas import tpu_sc as plsc

EMBED = 32
NC, NS = 2, 16
NW = NC * NS
BW = 128
BWP = BW + 1
K = 2
KG = 8
L = 16


def _format_sc(tableT, tail16):
    E, V = tableT.shape
    n_full = V // BW
    SW = 4
    n_sup = n_full // SW
    slots = 62
    mesh = plsc.VectorSubcoreMesh(core_axis_name="c", subcore_axis_name="s")

    @functools.partial(
        pl.kernel,
        out_type=jax.ShapeDtypeStruct((V * E // BW, BW), jnp.float32),
        mesh=mesh,
        compiler_params=pltpu.CompilerParams(
            use_tc_tiling_on_sc=True, needs_layout_passes=False),
        scratch_types=(
            [pltpu.VMEM((E, SW * BW), jnp.float32) for _ in range(K)]
            + [pltpu.VMEM((SW * BW * (E + 1),), jnp.float32) for _ in range(K)]
            + [pltpu.VMEM((SW * E, BW), jnp.float32) for _ in range(K)]
            + [pltpu.SemaphoreType.DMA for _ in range(2 * K)]
        ),
    )
    def k(tableT_hbm, tail_hbm, out_hbm, *scr):
        sbufs = scr[:K]
        dbvs = scr[K:2 * K]
        dbrs = scr[2 * K:3 * K]
        gsem = scr[3 * K:4 * K]
        wsem = scr[4 * K:]
        wid = lax.axis_index("s") * NC + lax.axis_index("c")
        lanes = lax.iota(jnp.int32, L)
        i33 = lanes * (E + 1)

        def s_of(j):
            return j * NW + wid

        def in_bounds(j):
            return s_of(j) < n_sup

        def fire_in(b, j):
            def _():
                pltpu.async_copy(
                    tableT_hbm.at[:, pl.ds(s_of(j) * SW * BW, SW * BW)],
                    sbufs[b], gsem[b])
            pl.when(in_bounds(j))(_)

        def wait_in(b, j):
            def _():
                pltpu.make_async_copy(
                    tableT_hbm.at[:, pl.ds(0, SW * BW)],
                    sbufs[b], gsem[b]).wait()
            pl.when(in_bounds(j))(_)

        def drain_out(b, j):
            def _():
                pltpu.make_async_copy(
                    dbrs[b], out_hbm.at[pl.ds(0, SW * E)], wsem[b]).wait()
            pl.when(in_bounds(j))(_)

        def transpose(b):
            @plsc.parallel_loop(0, E, unroll=8)
            def _(e):
                for kk in range(SW):
                    for v0 in range(0, BW, L):
                        vec = sbufs[b][e, pl.ds(kk * BW + v0, L)]
                        plsc.store_scatter(
                            dbvs[b],
                            [i33 + (kk * BW * (E + 1) + v0 * (E + 1) + e)],
                            vec)
            @plsc.parallel_loop(0, SW * E, unroll=8)
            def _(r):
                kk = r // E
                rr = r - kk * E
                base = kk * BW * (E + 1)
                for sv in range(4):
                    for h in range(2):
                        vec = plsc.load_gather(
                            dbvs[b],
                            [lanes + (base + (4 * rr + sv) * (E + 1) + h * L)])
                        dbrs[b][r, pl.ds(sv * E + h * L, L)] = vec

        @pl.when(wid == 0)
        def _():
            pltpu.sync_copy(tail_hbm, sbufs[0].at[pl.ds(0, 16), pl.ds(0, BW)])
            pltpu.sync_copy(
                sbufs[0].at[pl.ds(0, 16), pl.ds(0, BW)],
                out_hbm.at[pl.ds(n_full * BW * E // BW, 16)])

        for b in range(K):
            fire_in(b, b)

        def body(i, carry):
            for b in range(K):
                j = 2 * i + b
                wait_in(b, j)
                pl.when(jnp.logical_and(j >= 2, in_bounds(j - 2)))(
                    lambda b=b: pltpu.make_async_copy(
                        dbrs[b], out_hbm.at[pl.ds(0, SW * E)], wsem[b]).wait())

                def work(b=b, j=j):
                    transpose(b)
                    pltpu.async_copy(
                        dbrs[b],
                        out_hbm.at[pl.ds(s_of(j) * SW * E, SW * E)], wsem[b])
                pl.when(in_bounds(j))(work)
                fire_in(b, j + 2)
            return carry

        lax.fori_loop(0, slots // 2, body, 0)
        for b in range(K):
            drain_out(b, slots - 2 + b)

    return k(tableT, tail16)


@jax.jit
def _gather_sc(xT, table):
    length, batch = xT.shape
    assert batch == NW * BW
    n_iters = length // KG
    assert length % KG == 0
    mesh = plsc.VectorSubcoreMesh(core_axis_name="c", subcore_axis_name="s")

    @functools.partial(
        pl.kernel,
        out_type=jax.ShapeDtypeStruct(
            (length, EMBED // 8, NW, 8, BW), jnp.float32),
        mesh=mesh,
        compiler_params=pltpu.CompilerParams(
            use_tc_tiling_on_sc=False, needs_layout_passes=False),
        scratch_types=(
            [pltpu.VMEM((length, BW), jnp.int32)]
            + [pltpu.VMEM((BW, EMBED), jnp.float32) for _ in range(KG)]
            + [pltpu.VMEM((EMBED // 8, 8, BWP), jnp.float32) for _ in range(KG)]
            + [pltpu.SemaphoreType.DMA for _ in range(2 * KG)]
        ),
    )
    def k(xT_hbm, table_hbm, out_hbm, idx_v, *bufs_sems):
        bufs = bufs_sems[:KG]
        tbufs = bufs_sems[KG:2 * KG]
        gsem = bufs_sems[2 * KG:3 * KG]
        wsem = bufs_sems[3 * KG:]
        wid = lax.axis_index("s") * NC + lax.axis_index("c")
        b0 = wid * BW
        pltpu.sync_copy(xT_hbm.at[:, pl.ds(b0, BW)], idx_v)

        lanes = lax.iota(jnp.int32, L)
        ebs = [(lanes + e0) // 8 for e0 in (0, L)]
        ess = [(lanes + e0) % 8 for e0 in (0, L)]

        def tbuf_window(b):
            return tbufs[b].at[:, :, pl.ds(0, BW)]

        def fire_gather(b, t):
            return pltpu.async_copy(
                table_hbm.at[idx_v.at[t]], bufs[b], gsem[b])

        for b in range(KG):
            fire_gather(b, b)

        def body(i, carry):
            for b in range(KG):
                t = i * KG + b
                pltpu.make_async_copy(
                    table_hbm.at[idx_v.at[0]], bufs[b], gsem[b]).wait()

                def drain_tbuf(b=b):
                    pltpu.make_async_copy(
                        tbuf_window(b), out_hbm.at[0, :, wid], wsem[b]).wait()
                pl.when(i != 0)(drain_tbuf)
                @plsc.parallel_loop(0, BW, unroll=8)
                def _(v):
                    vcol = jnp.full((L,), v, jnp.int32)
                    for h in range(2):
                        vec = bufs[b][v, pl.ds(h * L, L)]
                        plsc.store_scatter(
                            tbufs[b], [ebs[h], ess[h], vcol], vec)
                pltpu.async_copy(
                    tbuf_window(b), out_hbm.at[t, :, wid], wsem[b])

                @pl.when(t + KG < length)
                def _():
                    fire_gather(b, t + KG)
            return carry

        lax.fori_loop(0, n_iters, body, 0)
        for b in range(KG):
            pltpu.make_async_copy(
                tbuf_window(b), out_hbm.at[0, :, wid], wsem[b]).wait()

    return k(xT, table)


@jax.jit
def _impl(x, idx2vec):
    batch, length = x.shape
    vocab = idx2vec.shape[0]
    tail16 = idx2vec[(vocab // BW) * BW:].reshape(-1, BW)
    scratch = _format_sc(idx2vec.T, tail16)
    out5 = _gather_sc(x.T, scratch.reshape(vocab, EMBED))
    return out5.transpose(2, 4, 0, 1, 3).reshape(batch, length, EMBED)


def kernel(x, idx2vec):
    return _impl(x, idx2vec)
